# baseline (speedup 1.0000x reference)
import functools

import jax
import jax.numpy as jnp
from jax import lax
from jax.experimental import pallas as pl
from jax.experimental.pallas import tpu as pltpu
from jax.experimental.pallas import tpu_sc as plsc

_N_TABLES = 26
_VOCAB = 100000
_DIM = 32
_B = 4096
_P = 20

_NC = 2
_NS = 16
_NW = _NC * _NS
_UNITS_W = (_N_TABLES * _DIM) // _NW
_L = 16
_BC = 512
_NCH = _B // _BC
_GRP = _BC // _L


def _sc_kernel(idx_hbm, w_hbm, out_hbm, idx_v, plane_v, out_v, sem_a, sem_b):
    wid = lax.axis_index("s") * _NC + lax.axis_index("c")
    sems = (sem_a, sem_b)

    def unit_body(u, _):
        gu = wid * _UNITS_W + u
        t = gu // _DIM
        d = gu % _DIM

        pltpu.sync_copy(w_hbm.at[t, d], plane_v)

        def idx_copy(c, buf):
            return pltpu.make_async_copy(
                idx_hbm.at[t, :, pl.ds(c * _BC, _BC)], idx_v.at[buf], sems[buf]
            )

        def compute(c, buf):
            def group_body(g, _):
                b0 = g * _L
                accs = [jnp.zeros((_L,), jnp.float32) for _ in range(4)]
                for p in range(_P):
                    iv = idx_v[buf, p, pl.ds(b0, _L)]
                    accs[p % 4] = accs[p % 4] + plsc.load_gather(plane_v, (iv,))
                out_v[pl.ds(c * _BC + b0, _L)] = (accs[0] + accs[1]) + (
                    accs[2] + accs[3]
                )
                return ()

            lax.fori_loop(0, _GRP, group_body, (), unroll=8)

        idx_copy(0, 0).start()

        def pair_body(k, _):
            c = 2 * k
            idx_copy(c, 0).wait()
            idx_copy(c + 1, 1).start()
            compute(c, 0)
            idx_copy(c + 1, 1).wait()

            @pl.when(k < (_NCH // 2) - 1)
            def _():
                idx_copy(c + 2, 0).start()

            compute(c + 1, 1)
            return ()

        lax.fori_loop(0, _NCH // 2, pair_body, ())

        pltpu.sync_copy(out_v, out_hbm.at[t, d])
        return ()

    lax.fori_loop(0, _UNITS_W, unit_body, ())


@jax.jit
def _run(lS_i, W):
    w_t = jnp.transpose(W, (0, 2, 1))
    idx_t = jnp.transpose(lS_i.reshape(_N_TABLES, _B, _P), (0, 2, 1))
    mesh = plsc.VectorSubcoreMesh(core_axis_name="c", subcore_axis_name="s")
    f = pl.kernel(
        _sc_kernel,
        out_type=jax.ShapeDtypeStruct((_N_TABLES, _DIM, _B), jnp.float32),
        mesh=mesh,
        scratch_types=[
            pltpu.VMEM((2, _P, _BC), jnp.int32),
            pltpu.VMEM((_VOCAB,), jnp.float32),
            pltpu.VMEM((_B,), jnp.float32),
            pltpu.SemaphoreType.DMA,
            pltpu.SemaphoreType.DMA,
        ],
        compiler_params=pltpu.CompilerParams(
            use_tc_tiling_on_sc=True, needs_layout_passes=False
        ),
    )
    out_t = f(idx_t, w_t)
    return jnp.transpose(out_t, (0, 2, 1))


def kernel(lS_o, lS_i, W):
    del lS_o
    return _run(lS_i, W)


# speedup vs baseline: 1.0197x; 1.0197x over previous
import functools

import jax
import jax.numpy as jnp
from jax import lax
from jax.experimental import pallas as pl
from jax.experimental.pallas import tpu as pltpu
from jax.experimental.pallas import tpu_sc as plsc

_N_TABLES = 26
_VOCAB = 100000
_DIM = 32
_B = 4096
_P = 20

_NC = 2
_NS = 16
_NW = _NC * _NS
_UNITS_W = (_N_TABLES * _DIM) // _NW
_L = 16
_BC = 512
_NCH = _B // _BC
_GRP = _BC // _L


def _sc_kernel(idx_hbm, w_hbm, out_hbm, idx_v, plane_v, out_v, sem_a, sem_b, sem_o):
    wid = lax.axis_index("s") * _NC + lax.axis_index("c")
    sems = (sem_a, sem_b)

    def unit_body(u, _):
        gu = wid * _UNITS_W + u
        t = gu // _DIM
        d = gu % _DIM

        def idx_copy(c, buf):
            return pltpu.make_async_copy(
                idx_hbm.at[t, :, pl.ds(c * _BC, _BC)], idx_v.at[buf], sems[buf]
            )

        idx_copy(0, 0).start()

        pltpu.sync_copy(w_hbm.at[t, d], plane_v)

        @pl.when(u > 0)
        def _():
            gp = gu - 1
            pltpu.make_async_copy(
                out_v, out_hbm.at[gp // _DIM, gp % _DIM], sem_o
            ).wait()

        def compute(c, buf):
            def group_body(g, _):
                b0 = g * _L
                accs = [jnp.zeros((_L,), jnp.float32) for _ in range(4)]
                for p in range(_P):
                    iv = idx_v[buf, p, pl.ds(b0, _L)]
                    accs[p % 4] = accs[p % 4] + plsc.load_gather(plane_v, (iv,))
                out_v[pl.ds(c * _BC + b0, _L)] = (accs[0] + accs[1]) + (
                    accs[2] + accs[3]
                )
                return ()

            lax.fori_loop(0, _GRP, group_body, (), unroll=4)

        def pair_body(k, _):
            c = 2 * k
            idx_copy(c, 0).wait()
            idx_copy(c + 1, 1).start()
            compute(c, 0)
            idx_copy(c + 1, 1).wait()

            @pl.when(k < (_NCH // 2) - 1)
            def _():
                idx_copy(c + 2, 0).start()

            compute(c + 1, 1)
            return ()

        lax.fori_loop(0, _NCH // 2, pair_body, ())

        pltpu.async_copy(out_v, out_hbm.at[t, d], sem_o)

        @pl.when(u == _UNITS_W - 1)
        def _():
            pltpu.make_async_copy(out_v, out_hbm.at[t, d], sem_o).wait()

        return ()

    lax.fori_loop(0, _UNITS_W, unit_body, ())


@jax.jit
def _run(lS_i, W):
    w_t = jnp.transpose(W, (0, 2, 1))
    idx_t = jnp.transpose(lS_i.reshape(_N_TABLES, _B, _P), (0, 2, 1))
    mesh = plsc.VectorSubcoreMesh(core_axis_name="c", subcore_axis_name="s")
    f = pl.kernel(
        _sc_kernel,
        out_type=jax.ShapeDtypeStruct((_N_TABLES, _DIM, _B), jnp.float32),
        mesh=mesh,
        scratch_types=[
            pltpu.VMEM((2, _P, _BC), jnp.int32),
            pltpu.VMEM((_VOCAB,), jnp.float32),
            pltpu.VMEM((_B,), jnp.float32),
            pltpu.SemaphoreType.DMA,
            pltpu.SemaphoreType.DMA,
            pltpu.SemaphoreType.DMA,
        ],
        compiler_params=pltpu.CompilerParams(
            use_tc_tiling_on_sc=True, needs_layout_passes=False
        ),
    )
    out_t = f(idx_t, w_t)
    return jnp.transpose(out_t, (0, 2, 1))


def kernel(lS_o, lS_i, W):
    del lS_o
    return _run(lS_i, W)
